# baseline (speedup 1.0000x reference)
import functools

import jax
import jax.numpy as jnp
from jax import lax
from jax.experimental import pallas as pl
from jax.experimental.pallas import tpu as pltpu, tpu_sc as plsc

NC = 2
NS = 16
NW = NC * NS
K = 128
WCH = 80


def _prep_body(x_ref, w_ref, y_ref):
    y_ref[...] = jnp.dot(x_ref[...], w_ref[0], preferred_element_type=jnp.float32)


def _prep(x, wd, tn=1000):
    n, d = x.shape
    r = wd.shape[0]
    nt = n // tn
    return pl.pallas_call(
        _prep_body,
        grid=(r, nt),
        in_specs=[
            pl.BlockSpec((tn, d), lambda ri, i: (i, 0)),
            pl.BlockSpec((1, d, d), lambda ri, i: (ri, 0, 0)),
        ],
        out_specs=pl.BlockSpec((tn, d), lambda ri, i: (ri * nt + i, 0)),
        out_shape=jax.ShapeDtypeStruct((r * n, d), jnp.float32),
    )(x, wd)


def _combine_body(p_ref, x_ref, lw_ref, b_ref, o_ref, *, act):
    out = (p_ref[0] + p_ref[1]
           + jnp.dot(x_ref[...], lw_ref[...], preferred_element_type=jnp.float32)
           + b_ref[...])
    if act:
        out = jnp.maximum(out, 0.0)
    o_ref[...] = out


def _combine(parts, x, loop_w, bias, act, tn=1000):
    n, d = x.shape
    nt = n // tn
    return pl.pallas_call(
        functools.partial(_combine_body, act=act),
        grid=(nt,),
        in_specs=[
            pl.BlockSpec((2, tn, d), lambda i: (0, i, 0)),
            pl.BlockSpec((tn, d), lambda i: (i, 0)),
            pl.BlockSpec((d, d), lambda i: (0, 0)),
            pl.BlockSpec((1, d), lambda i: (0, 0)),
        ],
        out_specs=pl.BlockSpec((tn, d), lambda i: (i, 0)),
        out_shape=jax.ShapeDtypeStruct((n, d), jnp.float32),
    )(parts, x, loop_w, bias.reshape(1, d))


CORE0_FRAC = 28


def _sc_edge(y, comb, normx, n, d):
    tot_ch = comb.shape[0]
    chunks_pair = tot_ch // NS
    chunks0 = (chunks_pair * CORE0_FRAC // 80 + 3) // 4 * 4
    chunks1 = chunks_pair - chunks0
    assert chunks0 % 4 == 0 and chunks1 % 4 == 0
    nch = n // WCH
    dg = d // 16
    mesh = plsc.VectorSubcoreMesh(core_axis_name="c", subcore_axis_name="s")

    def body(y_hbm, comb_hbm, normx_hbm, p_hbm,
             acc, cb0, cb1, cb2, cb3, nx, rows0, rows1,
             semC0, semC1, semC2, semC3, semN,
             semG0, semG1, semS0, semS1):
        cb = (cb0, cb1, cb2, cb3)
        rows = (rows0, rows1)
        semC = (semC0, semC1, semC2, semC3)
        semG = (semG0, semG1)
        semS = (semS0, semS1)
        cid = lax.axis_index("c")
        sid = lax.axis_index("s")
        chunks = jnp.where(cid == 0, chunks0, chunks1)
        ch_base = jnp.where(cid == 0, sid * chunks0,
                            NS * chunks0 + sid * chunks1)
        nk = (nch - sid + NS - 1) // NS

        zero16 = jnp.zeros((16,), jnp.float32)

        @plsc.parallel_loop(0, WCH, 1, unroll=4)
        def zrow(i):
            for c8 in range(dg):
                rows0[i, pl.ds(c8 * 16, 16)] = zero16

        def zchunk(i, _):
            k = sid + i * NS
            pltpu.sync_copy(rows0.at[pl.ds(0, WCH)], acc.at[pl.ds(k * WCH, WCH)])
            return 0

        lax.fori_loop(0, nk, zchunk, 0)
        plsc.subcore_barrier()

        def issue_cb(ch, qb):
            pltpu.async_copy(comb_hbm.at[ch_base + ch], cb[qb], semC[qb])

        def issue_nx(ch):
            pltpu.async_copy(normx_hbm.at[pl.ds((ch_base + ch) * K, K)], nx, semN)

        def wait_cb(qb):
            pltpu.make_async_copy(comb_hbm.at[0], cb[qb], semC[qb]).wait()

        def wait_nx():
            pltpu.make_async_copy(normx_hbm.at[pl.ds(0, K)], nx, semN).wait()

        def issue_gather(qb, rb):
            pltpu.async_copy(y_hbm.at[cb[qb].at[0]], rows[rb], semG[rb])

        def wait_gather(rb):
            pltpu.make_async_copy(y_hbm.at[cb[0].at[0]], rows[rb], semG[rb]).wait()

        def issue_scatter(qb, rb):
            pltpu.async_copy(rows[rb], acc.at[cb[qb].at[1]], semS[rb], add=True)

        def wait_scatter(rb):
            pltpu.make_async_copy(rows[rb], acc.at[cb[0].at[1]], semS[rb]).wait()

        def step(ch, i):
            rb = i % 2
            qb = i % 4
            wait_gather(rb)
            wait_nx()

            @plsc.parallel_loop(0, K, 1, unroll=4)
            def scale(j):
                nv = nx[j, :]
                for c8 in range(dg):
                    rows[rb][j, pl.ds(c8 * 16, 16)] = (
                        rows[rb][j, pl.ds(c8 * 16, 16)] * nv)

            issue_scatter(qb, rb)

            @pl.when(ch + 1 < chunks)
            def _():
                issue_nx(ch + 1)

            @pl.when(ch + 2 < chunks)
            def _():
                issue_cb(ch + 2, (qb + 2) % 4)

            @pl.when(ch + 1 < chunks)
            def _():
                wait_cb((qb + 1) % 4)

                @pl.when(ch >= 1)
                def _():
                    wait_scatter(1 - rb)
                issue_gather((qb + 1) % 4, 1 - rb)

        issue_cb(0, 0)
        issue_cb(1, 1)
        issue_nx(0)
        wait_cb(0)
        issue_gather(0, 0)

        def quad(g, _):
            for i in range(4):
                step(g * 4 + i, i)
            return 0

        lax.fori_loop(0, chunks // 4, quad, 0)
        wait_scatter(0)
        wait_scatter(1)
        plsc.subcore_barrier()

        def wchunk(i, _):
            k = sid + i * NS
            pltpu.sync_copy(acc.at[pl.ds(k * WCH, WCH)], rows0.at[pl.ds(0, WCH)])
            pltpu.sync_copy(rows0.at[pl.ds(0, WCH)],
                            p_hbm.at[cid, pl.ds(k * WCH, WCH)])
            return 0

        lax.fori_loop(0, nk, wchunk, 0)

    run = pl.kernel(
        body,
        out_type=jax.ShapeDtypeStruct((NC, n, d), jnp.float32),
        mesh=mesh,
        scratch_types=[
            pltpu.VMEM_SHARED((n, d), jnp.float32),
            pltpu.VMEM((2, K), jnp.int32), pltpu.VMEM((2, K), jnp.int32),
            pltpu.VMEM((2, K), jnp.int32), pltpu.VMEM((2, K), jnp.int32),
            pltpu.VMEM((K, 16), jnp.float32),
            pltpu.VMEM((K, d), jnp.float32), pltpu.VMEM((K, d), jnp.float32),
            pltpu.SemaphoreType.DMA, pltpu.SemaphoreType.DMA,
            pltpu.SemaphoreType.DMA, pltpu.SemaphoreType.DMA,
            pltpu.SemaphoreType.DMA, pltpu.SemaphoreType.DMA,
            pltpu.SemaphoreType.DMA, pltpu.SemaphoreType.DMA,
            pltpu.SemaphoreType.DMA,
        ],
    )
    return run(y, comb, normx)


def _block_diag_dense(w):
    r, b, blk, _ = w.shape
    d = b * blk
    eye = jnp.eye(b, dtype=w.dtype)
    wd = jnp.einsum('rbio,bc->rbico', w, eye).reshape(r, d, d)
    return wd


def _layer(x, comb, normx, wd, loop_w, bias, act):
    n, d = x.shape
    y = _prep(x, wd)
    parts = _sc_edge(y, comb, normx, n, d)
    return _combine(parts, x, loop_w, bias, act)


def kernel(h, edge_index, r, norm, emb_table, W1, loop_w1, bias1, W2, loop_w2, bias2):
    n, d = emb_table.shape
    e = edge_index.shape[1]

    x = emb_table[h]

    src = edge_index[0].astype(jnp.int32)
    dst = edge_index[1].astype(jnp.int32)
    g = r.astype(jnp.int32) * n + src
    e_pad = ((e + 4 * NW * K - 1) // (4 * NW * K)) * (4 * NW * K)
    pad = e_pad - e
    g = jnp.pad(g, (0, pad))
    dst_p = jnp.pad(dst, (0, pad))
    comb = jnp.stack([g.reshape(-1, K), dst_p.reshape(-1, K)], axis=1)
    normx = jnp.pad(norm.reshape(e, 1), ((0, pad), (0, 0)))
    normx = (normx * jnp.ones((1, 16), jnp.float32)).astype(jnp.float32)

    wd1 = _block_diag_dense(W1)
    wd2 = _block_diag_dense(W2)

    x1 = _layer(x, comb, normx, wd1, loop_w1, bias1, True)
    x2 = _layer(x1, comb, normx, wd2, loop_w2, bias2, False)
    return x2


# speedup vs baseline: 1.1011x; 1.1011x over previous
import functools

import jax
import jax.numpy as jnp
from jax import lax
from jax.experimental import pallas as pl
from jax.experimental.pallas import tpu as pltpu, tpu_sc as plsc

NC = 2
NS = 16
NW = NC * NS
K = 128
WCH = 80


def _prep_body(x_ref, w_ref, y_ref):
    y_ref[...] = jnp.dot(x_ref[...], w_ref[0], preferred_element_type=jnp.float32)


def _prep(x, wd, tn=1000):
    n, d = x.shape
    r = wd.shape[0]
    nt = n // tn
    return pl.pallas_call(
        _prep_body,
        grid=(r, nt),
        in_specs=[
            pl.BlockSpec((tn, d), lambda ri, i: (i, 0)),
            pl.BlockSpec((1, d, d), lambda ri, i: (ri, 0, 0)),
        ],
        out_specs=pl.BlockSpec((tn, d), lambda ri, i: (ri * nt + i, 0)),
        out_shape=jax.ShapeDtypeStruct((r * n, d), jnp.float32),
    )(x, wd)


def _combine_body(p_ref, x_ref, lw_ref, b_ref, o_ref, *, act):
    out = (p_ref[0] + p_ref[1]
           + jnp.dot(x_ref[...], lw_ref[...], preferred_element_type=jnp.float32)
           + b_ref[...])
    if act:
        out = jnp.maximum(out, 0.0)
    o_ref[...] = out


def _combine(parts, x, loop_w, bias, act, tn=1000):
    n, d = x.shape
    nt = n // tn
    return pl.pallas_call(
        functools.partial(_combine_body, act=act),
        grid=(nt,),
        in_specs=[
            pl.BlockSpec((2, tn, d), lambda i: (0, i, 0)),
            pl.BlockSpec((tn, d), lambda i: (i, 0)),
            pl.BlockSpec((d, d), lambda i: (0, 0)),
            pl.BlockSpec((1, d), lambda i: (0, 0)),
        ],
        out_specs=pl.BlockSpec((tn, d), lambda i: (i, 0)),
        out_shape=jax.ShapeDtypeStruct((n, d), jnp.float32),
    )(parts, x, loop_w, bias.reshape(1, d))


CORE0_FRAC = 52


def _sc_edge(y, comb, normx, n, d):
    tot_ch = comb.shape[0]
    chunks_pair = tot_ch // NS
    chunks0 = (chunks_pair * CORE0_FRAC // 80 + 3) // 4 * 4
    chunks1 = chunks_pair - chunks0
    assert chunks0 % 4 == 0 and chunks1 % 4 == 0
    nch = n // WCH
    dg = d // 16
    mesh = plsc.VectorSubcoreMesh(core_axis_name="c", subcore_axis_name="s")

    def body(y_hbm, comb_hbm, normx_hbm, p_hbm,
             acc, cb0, cb1, cb2, cb3, nx, rows0, rows1,
             semC0, semC1, semC2, semC3, semN,
             semG0, semG1, semS0, semS1):
        cb = (cb0, cb1, cb2, cb3)
        rows = (rows0, rows1)
        semC = (semC0, semC1, semC2, semC3)
        semG = (semG0, semG1)
        semS = (semS0, semS1)
        cid = lax.axis_index("c")
        sid = lax.axis_index("s")
        chunks = jnp.where(cid == 0, chunks0, chunks1)
        ch_base = jnp.where(cid == 0, sid * chunks0,
                            NS * chunks0 + sid * chunks1)
        nk = (nch - sid + NS - 1) // NS

        zero16 = jnp.zeros((16,), jnp.float32)

        @plsc.parallel_loop(0, WCH, 1, unroll=4)
        def zrow(i):
            for c8 in range(dg):
                rows0[i, pl.ds(c8 * 16, 16)] = zero16

        def zchunk(i, _):
            k = sid + i * NS
            pltpu.sync_copy(rows0.at[pl.ds(0, WCH)], acc.at[pl.ds(k * WCH, WCH)])
            return 0

        lax.fori_loop(0, nk, zchunk, 0)
        plsc.subcore_barrier()

        def issue_cb(ch, qb):
            pltpu.async_copy(comb_hbm.at[ch_base + ch], cb[qb], semC[qb])

        def issue_nx(ch):
            pltpu.async_copy(normx_hbm.at[pl.ds((ch_base + ch) * K, K)], nx, semN)

        def wait_cb(qb):
            pltpu.make_async_copy(comb_hbm.at[0], cb[qb], semC[qb]).wait()

        def wait_nx():
            pltpu.make_async_copy(normx_hbm.at[pl.ds(0, K)], nx, semN).wait()

        def issue_gather(qb, rb):
            pltpu.async_copy(y_hbm.at[cb[qb].at[0]], rows[rb], semG[rb])

        def wait_gather(rb):
            pltpu.make_async_copy(y_hbm.at[cb[0].at[0]], rows[rb], semG[rb]).wait()

        def issue_scatter(qb, rb):
            pltpu.async_copy(rows[rb], acc.at[cb[qb].at[1]], semS[rb], add=True)

        def wait_scatter(rb):
            pltpu.make_async_copy(rows[rb], acc.at[cb[0].at[1]], semS[rb]).wait()

        def step(ch, i):
            rb = i % 2
            qb = i % 4
            wait_gather(rb)
            wait_nx()

            @plsc.parallel_loop(0, K, 1, unroll=4)
            def scale(j):
                nv = nx[j, :]
                for c8 in range(dg):
                    rows[rb][j, pl.ds(c8 * 16, 16)] = (
                        rows[rb][j, pl.ds(c8 * 16, 16)] * nv)

            issue_scatter(qb, rb)

            @pl.when(ch + 1 < chunks)
            def _():
                issue_nx(ch + 1)

            @pl.when(ch + 2 < chunks)
            def _():
                issue_cb(ch + 2, (qb + 2) % 4)

            @pl.when(ch + 1 < chunks)
            def _():
                wait_cb((qb + 1) % 4)

                @pl.when(ch >= 1)
                def _():
                    wait_scatter(1 - rb)
                issue_gather((qb + 1) % 4, 1 - rb)

        issue_cb(0, 0)
        issue_cb(1, 1)
        issue_nx(0)
        wait_cb(0)
        issue_gather(0, 0)

        def quad(g, _):
            for i in range(4):
                step(g * 4 + i, i)
            return 0

        lax.fori_loop(0, chunks // 4, quad, 0)
        wait_scatter(0)
        wait_scatter(1)
        plsc.subcore_barrier()

        def wchunk(i, _):
            k = sid + i * NS
            pltpu.sync_copy(acc.at[pl.ds(k * WCH, WCH)], rows0.at[pl.ds(0, WCH)])
            pltpu.sync_copy(rows0.at[pl.ds(0, WCH)],
                            p_hbm.at[cid, pl.ds(k * WCH, WCH)])
            return 0

        lax.fori_loop(0, nk, wchunk, 0)

    run = pl.kernel(
        body,
        out_type=jax.ShapeDtypeStruct((NC, n, d), jnp.float32),
        mesh=mesh,
        scratch_types=[
            pltpu.VMEM_SHARED((n, d), jnp.float32),
            pltpu.VMEM((2, K), jnp.int32), pltpu.VMEM((2, K), jnp.int32),
            pltpu.VMEM((2, K), jnp.int32), pltpu.VMEM((2, K), jnp.int32),
            pltpu.VMEM((K, 16), jnp.float32),
            pltpu.VMEM((K, d), jnp.float32), pltpu.VMEM((K, d), jnp.float32),
            pltpu.SemaphoreType.DMA, pltpu.SemaphoreType.DMA,
            pltpu.SemaphoreType.DMA, pltpu.SemaphoreType.DMA,
            pltpu.SemaphoreType.DMA, pltpu.SemaphoreType.DMA,
            pltpu.SemaphoreType.DMA, pltpu.SemaphoreType.DMA,
            pltpu.SemaphoreType.DMA,
        ],
    )
    return run(y, comb, normx)


def _block_diag_dense(w):
    r, b, blk, _ = w.shape
    d = b * blk
    eye = jnp.eye(b, dtype=w.dtype)
    wd = jnp.einsum('rbio,bc->rbico', w, eye).reshape(r, d, d)
    return wd


def _layer(x, comb, normx, wd, loop_w, bias, act):
    n, d = x.shape
    y = _prep(x, wd)
    parts = _sc_edge(y, comb, normx, n, d)
    return _combine(parts, x, loop_w, bias, act)


def kernel(h, edge_index, r, norm, emb_table, W1, loop_w1, bias1, W2, loop_w2, bias2):
    n, d = emb_table.shape
    e = edge_index.shape[1]

    x = emb_table[h]

    src = edge_index[0].astype(jnp.int32)
    dst = edge_index[1].astype(jnp.int32)
    g = r.astype(jnp.int32) * n + src
    e_pad = ((e + 4 * NW * K - 1) // (4 * NW * K)) * (4 * NW * K)
    pad = e_pad - e
    g = jnp.pad(g, (0, pad))
    dst_p = jnp.pad(dst, (0, pad))
    comb = jnp.stack([g.reshape(-1, K), dst_p.reshape(-1, K)], axis=1)
    normx = jnp.pad(norm.reshape(e, 1), ((0, pad), (0, 0)))
    normx = (normx * jnp.ones((1, 16), jnp.float32)).astype(jnp.float32)

    wd1 = _block_diag_dense(W1)
    wd2 = _block_diag_dense(W2)

    x1 = _layer(x, comb, normx, wd1, loop_w1, bias1, True)
    x2 = _layer(x1, comb, normx, wd2, loop_w2, bias2, False)
    return x2
